# baseline (speedup 1.0000x reference)
import functools

import jax
import jax.numpy as jnp
from jax import lax
from jax.experimental import pallas as pl
from jax.experimental.pallas import tpu as pltpu
from jax.experimental.pallas import tpu_sc as plsc


def _take_last_body(x_hbm, seq_hbm, out_hbm, seq_s, sem, *, B, T):
    pltpu.sync_copy(seq_hbm, seq_s)
    copies = []
    for b in range(B):
        row = b * T + seq_s[b] - 1
        copies.append(pltpu.async_copy(x_hbm.at[row], out_hbm.at[b], sem))
    for cp in copies:
        cp.wait()


def kernel(x, seq_len):
    B, T, D = x.shape
    xf = x.reshape(B * T, D)
    seq = seq_len.astype(jnp.int32)
    mesh = plsc.ScalarSubcoreMesh(axis_name="c", num_cores=1)
    f = pl.kernel(
        functools.partial(_take_last_body, B=B, T=T),
        mesh=mesh,
        out_type=jax.ShapeDtypeStruct((B, D), jnp.float32),
        scratch_types=[
            pltpu.SMEM((B,), jnp.int32),
            pltpu.SemaphoreType.DMA,
        ],
    )
    return f(xf, seq)


# speedup vs baseline: 1.0016x; 1.0016x over previous
import functools

import jax
import jax.numpy as jnp
from jax.experimental import pallas as pl
from jax.experimental.pallas import tpu as pltpu
from jax.experimental.pallas import tpu_sc as plsc


def _take_last_body(x_hbm, seq_hbm, out_hbm, seq_s, sem, *, B, T):
    pltpu.sync_copy(seq_hbm, seq_s)
    copies = []
    for b in range(B):
        row = b * T + seq_s[b] - 1
        copies.append(pltpu.async_copy(x_hbm.at[row], out_hbm.at[b], sem))
    for cp in copies:
        cp.wait()


def kernel(x, seq_len):
    B, T, D = x.shape
    xf = x.reshape(B * T, D)
    seq = seq_len.astype(jnp.int32)
    mesh = plsc.ScalarSubcoreMesh(axis_name="c", num_cores=1)
    f = pl.kernel(
        functools.partial(_take_last_body, B=B, T=T),
        mesh=mesh,
        out_type=jax.ShapeDtypeStruct((B, D), jnp.float32),
        scratch_types=[
            pltpu.SMEM((B,), jnp.int32),
            pltpu.SemaphoreType.DMA,
        ],
    )
    return f(xf, seq)
